# baseline (speedup 1.0000x reference)
import functools

import jax
import jax.numpy as jnp
from jax import lax
from jax.experimental import pallas as pl
from jax.experimental.pallas import tpu as pltpu, tpu_sc as plsc


@functools.lru_cache(maxsize=None)
def _make_probe(V, D, B):
    info = plsc.get_sparse_core_info()
    mesh = plsc.VectorSubcoreMesh(core_axis_name="c", subcore_axis_name="s")

    @functools.partial(
        pl.kernel,
        mesh=mesh,
        compiler_params=pltpu.CompilerParams(
            use_tc_tiling_on_sc=True, needs_layout_passes=False
        ),
        out_type=jax.ShapeDtypeStruct((B, D), jnp.float32),
        scratch_types=[
            pltpu.VMEM((8, D), jnp.float32),
            pltpu.SemaphoreType.DMA,
        ],
    )
    def k(table_hbm, idx_hbm, out_hbm, rows_v, sem):
        wid = lax.axis_index("s") * info.num_cores + lax.axis_index("c")
        pltpu.sync_copy(table_hbm.at[wid], rows_v)
        pltpu.sync_copy(rows_v, out_hbm.at[pl.ds(wid * 8, 8)])

    return k


@jax.jit
def kernel(source, hidden, cell, emb):
    V, D = emb.shape
    B = source.shape[0]
    table3 = emb.reshape(V // 8, 8, D)
    return _make_probe(V, D, B)(table3, source)
